# baseline (speedup 1.0000x reference)
import functools

import jax
import jax.numpy as jnp
from jax import lax
from jax.experimental import pallas as pl
from jax.experimental.pallas import tpu as pltpu
from jax.experimental.pallas import tpu_sc as plsc

_R = 4
_BN = 128
_BH = 512
_N_SC = 6144


def _pool_sc(x4, n_sc, c, s):
    info = plsc.get_sparse_core_info()
    nw = info.num_cores * info.num_subcores
    iters = n_sc // (_R * nw)
    npairs = iters // 2
    mesh = plsc.VectorSubcoreMesh(core_axis_name="c", subcore_axis_name="s")
    vecs = (_R * c) // 16

    @functools.partial(
        pl.kernel,
        mesh=mesh,
        out_type=jax.ShapeDtypeStruct((n_sc, c), jnp.float32),
        scratch_types=[
            pltpu.VMEM((2, s, _R, c), jnp.float32),
            pltpu.VMEM((2, _R, c), jnp.float32),
            pltpu.SemaphoreType.DMA,
            pltpu.SemaphoreType.DMA,
            pltpu.SemaphoreType.DMA,
        ],
    )
    def pool(x_hbm, out_hbm, slab_v, outst_v, sem0, sem1, sem_out):
        wid = lax.axis_index("s") * info.num_cores + lax.axis_index("c")
        sems = (sem0, sem1)

        def row0_of(ci):
            return (ci * nw + wid) * _R

        def issue(ci, b):
            row0 = row0_of(ci)
            for si in range(s):
                pltpu.async_copy(
                    x_hbm.at[si, pl.ds(row0, _R), :], slab_v.at[b, si], sems[b]
                )

        def drain(ci, b):
            row0 = row0_of(ci)
            for si in range(s):
                pltpu.make_async_copy(
                    x_hbm.at[si, pl.ds(row0, _R), :], slab_v.at[b, si], sems[b]
                ).wait()

        def compute(ci, b):
            row0 = row0_of(ci)

            def vec_body(v, _):
                r = v // (c // 16)
                k = (v % (c // 16)) * 16
                acc = slab_v[b, 0, r, pl.ds(k, 16)]
                for si in range(1, s):
                    acc = acc + slab_v[b, si, r, pl.ds(k, 16)]
                outst_v[b, r, pl.ds(k, 16)] = acc * (1.0 / s)
                return 0

            lax.fori_loop(0, vecs, vec_body, 0)
            pltpu.async_copy(
                outst_v.at[b], out_hbm.at[pl.ds(row0, _R), :], sem_out
            ).wait()

        issue(0, 0)

        def pair_body(j, _):
            c0 = 2 * j
            c1 = 2 * j + 1
            issue(c1, 1)
            drain(c0, 0)
            compute(c0, 0)

            @pl.when(j + 1 < npairs)
            def _():
                issue(c0 + 2, 0)

            drain(c1, 1)
            compute(c1, 1)
            return 0

        lax.fori_loop(0, npairs, pair_body, 0)

    return pool(x4)


def _main_body(x_ref, wc_ref, bc_ref, wr_ref, br_ref, cls_ref, reg_ref):
    s = x_ref.shape[0]
    xm = jnp.sum(x_ref[...], axis=0) * (1.0 / s)
    dn = (((1,), (1,)), ((), ()))
    cls_ref[...] = (
        lax.dot_general(wc_ref[...], xm, dn, preferred_element_type=jnp.float32)
        + bc_ref[...]
    )
    reg_ref[...] = (
        lax.dot_general(wr_ref[...], xm, dn, preferred_element_type=jnp.float32)
        + br_ref[...]
    )


def _heads_body(xm_ref, wc_ref, bc_ref, wr_ref, br_ref, cls_ref, reg_ref):
    xm = xm_ref[...]
    dn = (((1,), (1,)), ((), ()))
    cls_ref[...] = (
        lax.dot_general(wc_ref[...], xm, dn, preferred_element_type=jnp.float32)
        + bc_ref[...]
    )
    reg_ref[...] = (
        lax.dot_general(wr_ref[...], xm, dn, preferred_element_type=jnp.float32)
        + br_ref[...]
    )


def kernel(x, W_cls, b_cls, W_reg, b_reg):
    n, c, rh, rw = x.shape
    s = rh * rw
    k1 = W_cls.shape[0]
    k2 = W_reg.shape[0]
    x4 = x.transpose(2, 3, 0, 1).reshape(s, n, c)
    bc2 = b_cls.reshape(k1, 1)
    br2 = b_reg.reshape(k2, 1)

    xm_sc = _pool_sc(x4, _N_SC, c, s)

    off = _N_SC // _BN
    grid = (n - _N_SC + _BN - 1) // _BN
    cls_t, reg_t = pl.pallas_call(
        _main_body,
        grid=(grid,),
        in_specs=[
            pl.BlockSpec((s, _BN, c), lambda i: (0, i + off, 0)),
            pl.BlockSpec((k1, c), lambda i: (0, 0)),
            pl.BlockSpec((k1, 1), lambda i: (0, 0)),
            pl.BlockSpec((k2, c), lambda i: (0, 0)),
            pl.BlockSpec((k2, 1), lambda i: (0, 0)),
        ],
        out_specs=[
            pl.BlockSpec((k1, _BN), lambda i: (0, i + off)),
            pl.BlockSpec((k2, _BN), lambda i: (0, i + off)),
        ],
        out_shape=[
            jax.ShapeDtypeStruct((k1, n), jnp.float32),
            jax.ShapeDtypeStruct((k2, n), jnp.float32),
        ],
    )(x4, W_cls, bc2, W_reg, br2)

    cls_sc, reg_sc = pl.pallas_call(
        _heads_body,
        grid=(_N_SC // _BH,),
        in_specs=[
            pl.BlockSpec((_BH, c), lambda i: (i, 0)),
            pl.BlockSpec((k1, c), lambda i: (0, 0)),
            pl.BlockSpec((k1, 1), lambda i: (0, 0)),
            pl.BlockSpec((k2, c), lambda i: (0, 0)),
            pl.BlockSpec((k2, 1), lambda i: (0, 0)),
        ],
        out_specs=[
            pl.BlockSpec((k1, _BH), lambda i: (0, i)),
            pl.BlockSpec((k2, _BH), lambda i: (0, i)),
        ],
        out_shape=[
            jax.ShapeDtypeStruct((k1, _N_SC), jnp.float32),
            jax.ShapeDtypeStruct((k2, _N_SC), jnp.float32),
        ],
    )(xm_sc, W_cls, bc2, W_reg, br2)

    cls_t = lax.dynamic_update_slice(cls_t, cls_sc, (0, 0))
    reg_t = lax.dynamic_update_slice(reg_t, reg_sc, (0, 0))
    return (cls_t.T, reg_t.T)


# speedup vs baseline: 1.0778x; 1.0778x over previous
import functools

import jax
import jax.numpy as jnp
from jax import lax
from jax.experimental import pallas as pl
from jax.experimental.pallas import tpu as pltpu
from jax.experimental.pallas import tpu_sc as plsc

_R = 4
_BN = 128
_BH = 512
_N_SC = 1024


def _pool_sc(x4, n_sc, c, s):
    info = plsc.get_sparse_core_info()
    nw = info.num_cores * info.num_subcores
    iters = n_sc // (_R * nw)
    npairs = iters // 2
    mesh = plsc.VectorSubcoreMesh(core_axis_name="c", subcore_axis_name="s")
    vecs = (_R * c) // 16

    @functools.partial(
        pl.kernel,
        mesh=mesh,
        out_type=jax.ShapeDtypeStruct((n_sc, c), jnp.float32),
        scratch_types=[
            pltpu.VMEM((2, s, _R, c), jnp.float32),
            pltpu.VMEM((2, _R, c), jnp.float32),
            pltpu.SemaphoreType.DMA,
            pltpu.SemaphoreType.DMA,
            pltpu.SemaphoreType.DMA,
        ],
    )
    def pool(x_hbm, out_hbm, slab_v, outst_v, sem0, sem1, sem_out):
        wid = lax.axis_index("s") * info.num_cores + lax.axis_index("c")
        sems = (sem0, sem1)

        def row0_of(ci):
            return (ci * nw + wid) * _R

        def issue(ci, b):
            row0 = row0_of(ci)
            for si in range(s):
                pltpu.async_copy(
                    x_hbm.at[si, pl.ds(row0, _R), :], slab_v.at[b, si], sems[b]
                )

        def drain(ci, b):
            row0 = row0_of(ci)
            for si in range(s):
                pltpu.make_async_copy(
                    x_hbm.at[si, pl.ds(row0, _R), :], slab_v.at[b, si], sems[b]
                ).wait()

        def compute(ci, b):
            row0 = row0_of(ci)

            def vec_body(v, _):
                r = v // (c // 16)
                k = (v % (c // 16)) * 16
                acc = slab_v[b, 0, r, pl.ds(k, 16)]
                for si in range(1, s):
                    acc = acc + slab_v[b, si, r, pl.ds(k, 16)]
                outst_v[b, r, pl.ds(k, 16)] = acc * (1.0 / s)
                return 0

            lax.fori_loop(0, vecs, vec_body, 0)
            pltpu.async_copy(
                outst_v.at[b], out_hbm.at[pl.ds(row0, _R), :], sem_out
            ).wait()

        issue(0, 0)

        def pair_body(j, _):
            c0 = 2 * j
            c1 = 2 * j + 1
            issue(c1, 1)
            drain(c0, 0)
            compute(c0, 0)

            @pl.when(j + 1 < npairs)
            def _():
                issue(c0 + 2, 0)

            drain(c1, 1)
            compute(c1, 1)
            return 0

        lax.fori_loop(0, npairs, pair_body, 0)

    return pool(x4)


def _main_body(x_ref, wc_ref, bc_ref, wr_ref, br_ref, cls_ref, reg_ref):
    s = x_ref.shape[0]
    xm = jnp.sum(x_ref[...], axis=0) * (1.0 / s)
    dn = (((1,), (1,)), ((), ()))
    cls_ref[...] = (
        lax.dot_general(wc_ref[...], xm, dn, preferred_element_type=jnp.float32)
        + bc_ref[...]
    )
    reg_ref[...] = (
        lax.dot_general(wr_ref[...], xm, dn, preferred_element_type=jnp.float32)
        + br_ref[...]
    )


def _heads_body(xm_ref, wc_ref, bc_ref, wr_ref, br_ref, cls_ref, reg_ref):
    xm = xm_ref[...]
    dn = (((1,), (1,)), ((), ()))
    cls_ref[...] = (
        lax.dot_general(wc_ref[...], xm, dn, preferred_element_type=jnp.float32)
        + bc_ref[...]
    )
    reg_ref[...] = (
        lax.dot_general(wr_ref[...], xm, dn, preferred_element_type=jnp.float32)
        + br_ref[...]
    )


def kernel(x, W_cls, b_cls, W_reg, b_reg):
    n, c, rh, rw = x.shape
    s = rh * rw
    k1 = W_cls.shape[0]
    k2 = W_reg.shape[0]
    x4 = x.transpose(2, 3, 0, 1).reshape(s, n, c)
    bc2 = b_cls.reshape(k1, 1)
    br2 = b_reg.reshape(k2, 1)

    xm_sc = _pool_sc(x4, _N_SC, c, s)

    off = _N_SC // _BN
    grid = (n - _N_SC + _BN - 1) // _BN
    cls_t, reg_t = pl.pallas_call(
        _main_body,
        grid=(grid,),
        in_specs=[
            pl.BlockSpec((s, _BN, c), lambda i: (0, i + off, 0)),
            pl.BlockSpec((k1, c), lambda i: (0, 0)),
            pl.BlockSpec((k1, 1), lambda i: (0, 0)),
            pl.BlockSpec((k2, c), lambda i: (0, 0)),
            pl.BlockSpec((k2, 1), lambda i: (0, 0)),
        ],
        out_specs=[
            pl.BlockSpec((k1, _BN), lambda i: (0, i + off)),
            pl.BlockSpec((k2, _BN), lambda i: (0, i + off)),
        ],
        out_shape=[
            jax.ShapeDtypeStruct((k1, n), jnp.float32),
            jax.ShapeDtypeStruct((k2, n), jnp.float32),
        ],
    )(x4, W_cls, bc2, W_reg, br2)

    cls_sc, reg_sc = pl.pallas_call(
        _heads_body,
        grid=(_N_SC // _BH,),
        in_specs=[
            pl.BlockSpec((_BH, c), lambda i: (i, 0)),
            pl.BlockSpec((k1, c), lambda i: (0, 0)),
            pl.BlockSpec((k1, 1), lambda i: (0, 0)),
            pl.BlockSpec((k2, c), lambda i: (0, 0)),
            pl.BlockSpec((k2, 1), lambda i: (0, 0)),
        ],
        out_specs=[
            pl.BlockSpec((k1, _BH), lambda i: (0, i)),
            pl.BlockSpec((k2, _BH), lambda i: (0, i)),
        ],
        out_shape=[
            jax.ShapeDtypeStruct((k1, _N_SC), jnp.float32),
            jax.ShapeDtypeStruct((k2, _N_SC), jnp.float32),
        ],
    )(xm_sc, W_cls, bc2, W_reg, br2)

    cls_t = lax.dynamic_update_slice(cls_t, cls_sc, (0, 0))
    reg_t = lax.dynamic_update_slice(reg_t, reg_sc, (0, 0))
    return (cls_t.T, reg_t.T)


# speedup vs baseline: 1.0911x; 1.0124x over previous
import functools

import jax
import jax.numpy as jnp
from jax import lax
from jax.experimental import pallas as pl
from jax.experimental.pallas import tpu as pltpu
from jax.experimental.pallas import tpu_sc as plsc

_R = 4
_BN = 128
_BH = 512
_N_SC = 1024


def _pool_sc(x4, n_sc, c, s):
    info = plsc.get_sparse_core_info()
    nw = info.num_cores * info.num_subcores
    iters = n_sc // (_R * nw)
    npairs = iters // 2
    mesh = plsc.VectorSubcoreMesh(core_axis_name="c", subcore_axis_name="s")
    vecs = (_R * c) // 16

    @functools.partial(
        pl.kernel,
        mesh=mesh,
        out_type=jax.ShapeDtypeStruct((n_sc, c), jnp.float32),
        scratch_types=[
            pltpu.VMEM((2, s, _R, c), jnp.float32),
            pltpu.VMEM((2, _R, c), jnp.float32),
            pltpu.SemaphoreType.DMA,
            pltpu.SemaphoreType.DMA,
            pltpu.SemaphoreType.DMA,
        ],
    )
    def pool(x_hbm, out_hbm, slab_v, outst_v, sem0, sem1, sem_out):
        wid = lax.axis_index("s") * info.num_cores + lax.axis_index("c")
        sems = (sem0, sem1)

        def row0_of(ci):
            return (ci * nw + wid) * _R

        def issue(ci, b):
            row0 = row0_of(ci)
            for si in range(s):
                pltpu.async_copy(
                    x_hbm.at[si, pl.ds(row0, _R), :], slab_v.at[b, si], sems[b]
                )

        def drain(ci, b):
            row0 = row0_of(ci)
            for si in range(s):
                pltpu.make_async_copy(
                    x_hbm.at[si, pl.ds(row0, _R), :], slab_v.at[b, si], sems[b]
                ).wait()

        def compute(ci, b):
            row0 = row0_of(ci)

            def vec_body(v, _):
                r = v // (c // 16)
                k = (v % (c // 16)) * 16
                acc = slab_v[b, 0, r, pl.ds(k, 16)]
                for si in range(1, s):
                    acc = acc + slab_v[b, si, r, pl.ds(k, 16)]
                outst_v[b, r, pl.ds(k, 16)] = acc * (1.0 / s)
                return 0

            lax.fori_loop(0, vecs, vec_body, 0)
            pltpu.async_copy(
                outst_v.at[b], out_hbm.at[pl.ds(row0, _R), :], sem_out
            ).wait()

        issue(0, 0)

        def pair_body(j, _):
            c0 = 2 * j
            c1 = 2 * j + 1
            issue(c1, 1)
            drain(c0, 0)
            compute(c0, 0)

            @pl.when(j + 1 < npairs)
            def _():
                issue(c0 + 2, 0)

            drain(c1, 1)
            compute(c1, 1)
            return 0

        lax.fori_loop(0, npairs, pair_body, 0)

    return pool(x4)


def _main_body(x_ref, wc_ref, bc_ref, wr_ref, br_ref, cls_ref, reg_ref):
    s = x_ref.shape[0]
    xm = jnp.sum(x_ref[...], axis=0) * (1.0 / s)
    dn = (((1,), (1,)), ((), ()))
    cls_ref[...] = (
        lax.dot_general(wc_ref[...], xm, dn, preferred_element_type=jnp.float32)
        + bc_ref[...]
    )
    reg_ref[...] = (
        lax.dot_general(wr_ref[...], xm, dn, preferred_element_type=jnp.float32)
        + br_ref[...]
    )


def _heads_body(
    xm_ref, wc_ref, bc_ref, wr_ref, br_ref, cls_in, reg_in, cls_ref, reg_ref
):
    del cls_in, reg_in
    xm = xm_ref[...]
    dn = (((1,), (1,)), ((), ()))
    cls_ref[...] = (
        lax.dot_general(wc_ref[...], xm, dn, preferred_element_type=jnp.float32)
        + bc_ref[...]
    )
    reg_ref[...] = (
        lax.dot_general(wr_ref[...], xm, dn, preferred_element_type=jnp.float32)
        + br_ref[...]
    )


def kernel(x, W_cls, b_cls, W_reg, b_reg):
    n, c, rh, rw = x.shape
    s = rh * rw
    k1 = W_cls.shape[0]
    k2 = W_reg.shape[0]
    x4 = x.transpose(2, 3, 0, 1).reshape(s, n, c)
    bc2 = b_cls.reshape(k1, 1)
    br2 = b_reg.reshape(k2, 1)

    xm_sc = _pool_sc(x4, _N_SC, c, s)

    off = _N_SC // _BN
    grid = (n - _N_SC + _BN - 1) // _BN
    cls_t, reg_t = pl.pallas_call(
        _main_body,
        grid=(grid,),
        in_specs=[
            pl.BlockSpec((s, _BN, c), lambda i: (0, i + off, 0)),
            pl.BlockSpec((k1, c), lambda i: (0, 0)),
            pl.BlockSpec((k1, 1), lambda i: (0, 0)),
            pl.BlockSpec((k2, c), lambda i: (0, 0)),
            pl.BlockSpec((k2, 1), lambda i: (0, 0)),
        ],
        out_specs=[
            pl.BlockSpec((k1, _BN), lambda i: (0, i + off)),
            pl.BlockSpec((k2, _BN), lambda i: (0, i + off)),
        ],
        out_shape=[
            jax.ShapeDtypeStruct((k1, n), jnp.float32),
            jax.ShapeDtypeStruct((k2, n), jnp.float32),
        ],
    )(x4, W_cls, bc2, W_reg, br2)

    cls_t, reg_t = pl.pallas_call(
        _heads_body,
        grid=(_N_SC // _BH,),
        in_specs=[
            pl.BlockSpec((_BH, c), lambda i: (i, 0)),
            pl.BlockSpec((k1, c), lambda i: (0, 0)),
            pl.BlockSpec((k1, 1), lambda i: (0, 0)),
            pl.BlockSpec((k2, c), lambda i: (0, 0)),
            pl.BlockSpec((k2, 1), lambda i: (0, 0)),
            pl.BlockSpec(memory_space=pltpu.MemorySpace.HBM),
            pl.BlockSpec(memory_space=pltpu.MemorySpace.HBM),
        ],
        out_specs=[
            pl.BlockSpec((k1, _BH), lambda i: (0, i)),
            pl.BlockSpec((k2, _BH), lambda i: (0, i)),
        ],
        out_shape=[
            jax.ShapeDtypeStruct((k1, n), jnp.float32),
            jax.ShapeDtypeStruct((k2, n), jnp.float32),
        ],
        input_output_aliases={5: 0, 6: 1},
    )(xm_sc, W_cls, bc2, W_reg, br2, cls_t, reg_t)

    return (cls_t.T, reg_t.T)


# speedup vs baseline: 1.1639x; 1.0667x over previous
import jax
import jax.numpy as jnp
from jax import lax
from jax.experimental import pallas as pl

_BN = 256


def _body(x_ref, wc_ref, bc_ref, wr_ref, br_ref, cls_ref, reg_ref):
    s = x_ref.shape[0]
    xm = jnp.sum(x_ref[...], axis=0) * (1.0 / s)
    dn = (((1,), (1,)), ((), ()))
    cls_ref[...] = (
        lax.dot_general(wc_ref[...], xm, dn, preferred_element_type=jnp.float32)
        + bc_ref[...]
    )
    reg_ref[...] = (
        lax.dot_general(wr_ref[...], xm, dn, preferred_element_type=jnp.float32)
        + br_ref[...]
    )


def kernel(x, W_cls, b_cls, W_reg, b_reg):
    n, c, rh, rw = x.shape
    s = rh * rw
    k1 = W_cls.shape[0]
    k2 = W_reg.shape[0]
    x4 = x.transpose(2, 3, 0, 1).reshape(s, n, c)
    bc2 = b_cls.reshape(k1, 1)
    br2 = b_reg.reshape(k2, 1)
    grid = (n + _BN - 1) // _BN
    cls_t, reg_t = pl.pallas_call(
        _body,
        grid=(grid,),
        in_specs=[
            pl.BlockSpec((s, _BN, c), lambda i: (0, i, 0)),
            pl.BlockSpec((k1, c), lambda i: (0, 0)),
            pl.BlockSpec((k1, 1), lambda i: (0, 0)),
            pl.BlockSpec((k2, c), lambda i: (0, 0)),
            pl.BlockSpec((k2, 1), lambda i: (0, 0)),
        ],
        out_specs=[
            pl.BlockSpec((k1, _BN), lambda i: (0, i)),
            pl.BlockSpec((k2, _BN), lambda i: (0, i)),
        ],
        out_shape=[
            jax.ShapeDtypeStruct((k1, n), jnp.float32),
            jax.ShapeDtypeStruct((k2, n), jnp.float32),
        ],
    )(x4, W_cls, bc2, W_reg, br2)
    return (cls_t.T, reg_t.T)
